# baseline (speedup 1.0000x reference)
import functools

import jax
import jax.numpy as jnp
from jax import lax
from jax.experimental import pallas as pl
from jax.experimental.pallas import tpu as pltpu
from jax.experimental.pallas import tpu_sc as plsc

_EMBED = 64
_WORDS = _EMBED // 4
_CHUNK = 512


@functools.lru_cache(maxsize=None)
def _build(B, V):
    info = plsc.get_sparse_core_info()
    NC, NS, L = info.num_cores, info.num_subcores, info.num_lanes
    NW = NC * NS
    assert B % NW == 0
    b_per_w = B // NW
    assert b_per_w % _CHUNK == 0
    n_chunks = b_per_w // _CHUNK

    mesh = plsc.VectorSubcoreMesh(core_axis_name="c", subcore_axis_name="s")

    @functools.partial(
        pl.kernel,
        mesh=mesh,
        compiler_params=pltpu.CompilerParams(
            needs_layout_passes=False, use_tc_tiling_on_sc=False
        ),
        out_type=jax.ShapeDtypeStruct((B * _EMBED,), jnp.float32),
        scratch_types=[
            pltpu.VMEM((_CHUNK,), jnp.int32),
            pltpu.VMEM((_CHUNK, _WORDS), jnp.int32),
            pltpu.VMEM((_CHUNK,), jnp.float32),
            pltpu.VMEM((_CHUNK * _EMBED,), jnp.float32),
            pltpu.SemaphoreType.DMA,
            pltpu.SemaphoreType.DMA,
        ],
    )
    def k(idx_hbm, qw_hbm, sc_hbm, out_hbm, idx_v, rows_v, scl_v, out_v, sem_r, sem_s):
        wid = lax.axis_index("s") * NC + lax.axis_index("c")
        base = wid * b_per_w
        lanes4 = lax.iota(jnp.int32, L) * 4

        def chunk_body(c, _):
            off = base + c * _CHUNK
            pltpu.sync_copy(idx_hbm.at[pl.ds(off, _CHUNK)], idx_v)
            cp_rows = pltpu.async_copy(qw_hbm.at[idx_v], rows_v, sem_r)
            cp_scl = pltpu.async_copy(sc_hbm.at[idx_v], scl_v, sem_s)
            cp_rows.wait()
            cp_scl.wait()

            def group_body(g, _):
                g16 = g * L
                sv = scl_v[pl.ds(g16, L)]
                for i in range(L):
                    r = g16 + i
                    w = rows_v[r]
                    s = sv[i]
                    rbase = lanes4 + r * _EMBED
                    for b in range(4):
                        v = (w << (24 - 8 * b)) >> 24
                        f = v.astype(jnp.float32) * s
                        plsc.store_scatter(out_v, [rbase + b], f)
                return 0

            lax.fori_loop(0, _CHUNK // L, group_body, 0)
            pltpu.sync_copy(out_v, out_hbm.at[pl.ds(off * _EMBED, _CHUNK * _EMBED)])
            return 0

        lax.fori_loop(0, n_chunks, chunk_body, 0)

    return k


def kernel(input_ids, q_weight, scale):
    BATCH, HIST = input_ids.shape
    V, E = q_weight.shape
    B = BATCH * HIST
    idx = input_ids.reshape(B)
    qw_words = jax.lax.bitcast_convert_type(
        q_weight.reshape(V, _WORDS, 4), jnp.int32
    )
    out = _build(B, V)(idx, qw_words, scale.reshape(V))
    return out.reshape(BATCH, HIST, E)


# speedup vs baseline: 1.3563x; 1.3563x over previous
import functools

import jax
import jax.numpy as jnp
from jax import lax
from jax.experimental import pallas as pl
from jax.experimental.pallas import tpu as pltpu
from jax.experimental.pallas import tpu_sc as plsc

_EMBED = 64
_WORDS = _EMBED // 4
_CHUNK = 512


@functools.lru_cache(maxsize=None)
def _build(B, V):
    info = plsc.get_sparse_core_info()
    NC, NS, L = info.num_cores, info.num_subcores, info.num_lanes
    NW = NC * NS
    assert B % NW == 0
    b_per_w = B // NW
    assert b_per_w % _CHUNK == 0
    n_chunks = b_per_w // _CHUNK

    mesh = plsc.VectorSubcoreMesh(core_axis_name="c", subcore_axis_name="s")

    @functools.partial(
        pl.kernel,
        mesh=mesh,
        compiler_params=pltpu.CompilerParams(
            needs_layout_passes=False, use_tc_tiling_on_sc=False
        ),
        out_type=jax.ShapeDtypeStruct((B * _EMBED,), jnp.float32),
        scratch_types=[
            pltpu.VMEM((_CHUNK,), jnp.int32),
            pltpu.VMEM((_CHUNK, _EMBED), jnp.int8),
            pltpu.VMEM((_CHUNK,), jnp.float32),
            pltpu.VMEM((_CHUNK * _EMBED,), jnp.float32),
            pltpu.SemaphoreType.DMA,
            pltpu.SemaphoreType.DMA,
        ],
    )
    def k(idx_hbm, qw_hbm, sc_hbm, out_hbm, idx_v, rows_v, scl_v, out_v, sem_r, sem_s):
        wid = lax.axis_index("s") * NC + lax.axis_index("c")
        base = wid * b_per_w
        lanes4 = lax.iota(jnp.int32, L) * 4

        def chunk_body(c, _):
            off = base + c * _CHUNK
            pltpu.sync_copy(idx_hbm.at[pl.ds(off, _CHUNK)], idx_v)
            cp_rows = pltpu.async_copy(qw_hbm.at[idx_v], rows_v, sem_r)
            cp_scl = pltpu.async_copy(sc_hbm.at[idx_v], scl_v, sem_s)
            cp_rows.wait()
            cp_scl.wait()

            def group_body(g, _):
                g16 = g * L
                sv = scl_v[pl.ds(g16, L)]
                for i in range(L):
                    r = g16 + i
                    w = plsc.bitcast(rows_v[r], jnp.int32)
                    s = sv[i]
                    rbase = lanes4 + r * _EMBED
                    for b in range(4):
                        v = (w << (24 - 8 * b)) >> 24
                        f = v.astype(jnp.float32) * s
                        plsc.store_scatter(out_v, [rbase + b], f)
                return 0

            lax.fori_loop(0, _CHUNK // L, group_body, 0)
            pltpu.sync_copy(out_v, out_hbm.at[pl.ds(off * _EMBED, _CHUNK * _EMBED)])
            return 0

        lax.fori_loop(0, n_chunks, chunk_body, 0)

    return k


def kernel(input_ids, q_weight, scale):
    BATCH, HIST = input_ids.shape
    V, E = q_weight.shape
    B = BATCH * HIST
    idx = input_ids.reshape(B)
    out = _build(B, V)(idx, q_weight, scale.reshape(V))
    return out.reshape(BATCH, HIST, E)
